# baseline (speedup 1.0000x reference)
import functools

import jax
import jax.numpy as jnp
from jax import lax
from jax.experimental import pallas as pl
from jax.experimental.pallas import tpu as pltpu
from jax.experimental.pallas import tpu_sc as plsc

VOCAB = 100000
EMBED = 128
HIDDEN = 256
BATCH = 4096
SEQ = 50
TOKENS = BATCH * SEQ

_NS = 2
_STOK = TOKENS // _NS

_NW = 32
_TPW = _STOK // _NW
_CHUNK = 64
_NCHUNK = _TPW // _CHUNK
_NBUF = 2


def _gather_body(table_hbm, idx_hbm, out_hbm, idx_v, bufs, gsems, ssems):
    wid = lax.axis_index("s") * 2 + lax.axis_index("c")
    base = wid * _TPW

    pltpu.sync_copy(idx_hbm.at[wid], idx_v)

    def gather_op(chunk, b):
        return pltpu.make_async_copy(
            table_hbm.at[idx_v.at[chunk]], bufs[b], gsems[b])

    def scatter_op(chunk, b):
        off = base + chunk * _CHUNK
        return pltpu.make_async_copy(
            bufs[b], out_hbm.at[pl.ds(off, _CHUNK)], ssems[b])

    for b in range(_NBUF):
        gather_op(b, b).start()

    def group(g, carry):
        for b in range(_NBUF):
            i = g * _NBUF + b
            gather_op(i, b).wait()
            scatter_op(i, b).start()
            scatter_op(i, b).wait()
            gather_op(i + _NBUF, b).start()
        return carry

    lax.fori_loop(0, (_NCHUNK - _NBUF) // _NBUF, group, 0)

    for b in range(_NBUF):
        i = _NCHUNK - _NBUF + b
        gather_op(i, b).wait()
        scatter_op(i, b).start()
    for b in range(_NBUF):
        i = _NCHUNK - _NBUF + b
        scatter_op(i, b).wait()


_gather = functools.partial(
    pl.kernel,
    out_type=jax.ShapeDtypeStruct((_STOK, EMBED), jnp.float32),
    mesh=plsc.VectorSubcoreMesh(core_axis_name="c", subcore_axis_name="s"),
    scratch_types=[
        pltpu.VMEM((_NCHUNK, _CHUNK), jnp.int32),
        [pltpu.VMEM((_CHUNK, EMBED), jnp.float32) for _ in range(_NBUF)],
        [pltpu.SemaphoreType.DMA for _ in range(_NBUF)],
        [pltpu.SemaphoreType.DMA for _ in range(_NBUF)],
    ],
)(_gather_body)


_TB = 3200
_SSTEP = _STOK // _TB


def _proj_body(e_ref, w_ref, b_ref, o_ref):
    acc = jnp.dot(e_ref[...], w_ref[...], preferred_element_type=jnp.float32)
    o_ref[...] = jnp.maximum(acc + b_ref[...], 0.0)


def _proj_body_acc(e_ref, w_ref, b_ref, prev_ref, o_ref):
    del prev_ref
    _proj_body(e_ref, w_ref, b_ref, o_ref)


def _project_slice(embs, W, b, s, prev):
    common = dict(
        grid=(_SSTEP,),
        out_specs=pl.BlockSpec((_TB, HIDDEN), lambda i, s=s: (s * _SSTEP + i, 0)),
        out_shape=jax.ShapeDtypeStruct((TOKENS, HIDDEN), jnp.float32),
    )
    in_specs = [
        pl.BlockSpec((_TB, EMBED), lambda i: (i, 0)),
        pl.BlockSpec((EMBED, HIDDEN), lambda i: (0, 0)),
        pl.BlockSpec((1, HIDDEN), lambda i: (0, 0)),
    ]
    if prev is None:
        return pl.pallas_call(_proj_body, in_specs=in_specs, **common)(
            embs, W, b.reshape(1, HIDDEN))
    return pl.pallas_call(
        _proj_body_acc,
        in_specs=in_specs + [pl.BlockSpec(memory_space=pl.ANY)],
        input_output_aliases={3: 0},
        **common,
    )(embs, W, b.reshape(1, HIDDEN), prev)


def kernel(X, table, W, b):
    idx = X.T.reshape(_NS, _NW, _NCHUNK, _CHUNK).astype(jnp.int32)
    embs = [_gather(table, idx[s]) for s in range(_NS)]
    out = None
    for s in range(_NS):
        out = _project_slice(embs[s], W, b, s, out)
    return out.reshape(SEQ, BATCH, HIDDEN).transpose(1, 0, 2)


# speedup vs baseline: 1.0337x; 1.0337x over previous
import functools

import jax
import jax.numpy as jnp
from jax import lax
from jax.experimental import pallas as pl
from jax.experimental.pallas import tpu as pltpu
from jax.experimental.pallas import tpu_sc as plsc

VOCAB = 100000
EMBED = 128
HIDDEN = 256
BATCH = 4096
SEQ = 50
TOKENS = BATCH * SEQ

_NW = 32
_TPW = TOKENS // _NW
_CHUNK = 128
_NCHUNK = _TPW // _CHUNK
_NBUF = 2


def _gather_body(table_hbm, idx_hbm, out_hbm, idx_v, bufs, gsems, ssems):
    wid = lax.axis_index("s") * 2 + lax.axis_index("c")
    base = wid * _TPW

    pltpu.sync_copy(idx_hbm.at[wid], idx_v)

    def gather_op(chunk, b):
        return pltpu.make_async_copy(
            table_hbm.at[idx_v.at[chunk]], bufs[b], gsems[b])

    def scatter_op(chunk, b):
        off = base + chunk * _CHUNK
        return pltpu.make_async_copy(
            bufs[b], out_hbm.at[pl.ds(off, _CHUNK)], ssems[b])

    for b in range(_NBUF):
        gather_op(b, b).start()

    def group(g, carry):
        for b in range(_NBUF):
            i = g * _NBUF + b
            gather_op(i, b).wait()
            scatter_op(i, b).start()
            scatter_op(i, b).wait()
            gather_op(i + _NBUF, b).start()
        return carry

    lax.fori_loop(0, (_NCHUNK - _NBUF) // _NBUF, group, 0)

    for b in range(_NBUF):
        i = _NCHUNK - _NBUF + b
        gather_op(i, b).wait()
        scatter_op(i, b).start()
    for b in range(_NBUF):
        i = _NCHUNK - _NBUF + b
        scatter_op(i, b).wait()


_gather = functools.partial(
    pl.kernel,
    out_type=jax.ShapeDtypeStruct((TOKENS, EMBED), jnp.float32),
    mesh=plsc.VectorSubcoreMesh(core_axis_name="c", subcore_axis_name="s"),
    scratch_types=[
        pltpu.VMEM((_NCHUNK, _CHUNK), jnp.int32),
        [pltpu.VMEM((_CHUNK, EMBED), jnp.float32) for _ in range(_NBUF)],
        [pltpu.SemaphoreType.DMA for _ in range(_NBUF)],
        [pltpu.SemaphoreType.DMA for _ in range(_NBUF)],
    ],
)(_gather_body)


_TB = 3200


def _proj_body(e_ref, w_ref, b_ref, o_ref):
    acc = jnp.dot(e_ref[...], w_ref[...], preferred_element_type=jnp.float32)
    o_ref[...] = jnp.maximum(acc + b_ref[...], 0.0)


def _project(embs, W, b):
    return pl.pallas_call(
        _proj_body,
        grid=(TOKENS // _TB,),
        in_specs=[
            pl.BlockSpec((_TB, EMBED), lambda i: (i, 0)),
            pl.BlockSpec((EMBED, HIDDEN), lambda i: (0, 0)),
            pl.BlockSpec((1, HIDDEN), lambda i: (0, 0)),
        ],
        out_specs=pl.BlockSpec((_TB, HIDDEN), lambda i: (i, 0)),
        out_shape=jax.ShapeDtypeStruct((TOKENS, HIDDEN), jnp.float32),
    )(embs, W, b.reshape(1, HIDDEN))


def kernel(X, table, W, b):
    idx = X.T.reshape(_NW, _NCHUNK, _CHUNK).astype(jnp.int32)
    embs = _gather(table, idx)
    out = _project(embs, W, b)
    return out.reshape(SEQ, BATCH, HIDDEN).transpose(1, 0, 2)
